# Initial kernel scaffold; baseline (speedup 1.0000x reference)
import jax
import jax.numpy as jnp
from jax.experimental import pallas as pl


def kernel(x, edge_index0, edge_index1, W_l0, W_r0, b0, W_l1, W_r1, b1):
    raise NotImplementedError("write your pallas kernel here")


# speedup vs baseline: 5.1363x; 5.1363x over previous
import functools

import jax
import jax.numpy as jnp
from jax import lax
from jax.experimental import pallas as pl
from jax.experimental.pallas import tpu as pltpu
from jax.experimental.pallas import tpu_sc as plsc

_N1, _N2 = 20000, 4096
_E0, _E1 = 320000, 65536
_IN_C, _HID, _OUT_C = 128, 256, 47

_N1P = 20480
_STR0 = _N1P // 16
_C0 = 160
_E0P = _C0 * 16 * 128
_C0H = _C0 // 2
_C1 = 16
_STR1 = _N2 // 16

_mesh = plsc.VectorSubcoreMesh(core_axis_name="c", subcore_axis_name="s")
_sc_params = pltpu.CompilerParams(use_tc_tiling_on_sc=False)


@functools.partial(
    pl.kernel,
    out_type=[
        jax.ShapeDtypeStruct((2, _N1P, 16), jnp.float32),
        jax.ShapeDtypeStruct((2, _N2, 16), jnp.float32),
    ],
    mesh=_mesh,
    compiler_params=_sc_params,
    scratch_types=[
        pltpu.VMEM((_C0H, 128), jnp.int32),
        pltpu.VMEM((_C1, 128), jnp.int32),
        pltpu.VMEM((128, 16), jnp.float32),
        pltpu.VMEM_SHARED((_N1P, 16), jnp.float32),
        pltpu.VMEM_SHARED((_N2, 16), jnp.float32),
    ],
)
def _sc_counts(dst0c, dst1c, z0, z1, ones,
               cnt0_out, cnt1_out,
               d0, d1, ones_v, cnt0, cnt1):
    c = lax.axis_index("c")
    s = lax.axis_index("s")
    pltpu.sync_copy(z0, cnt0.at[pl.ds(s * _STR0, _STR0)])
    pltpu.sync_copy(z1, cnt1.at[pl.ds(s * _STR1, _STR1)])
    base0 = c * (_C0H * 16) + s * _C0H
    base1 = c * (_C1 * 16) + s * _C1
    pltpu.sync_copy(dst0c.at[pl.ds(base0, _C0H)], d0)
    pltpu.sync_copy(dst1c.at[pl.ds(base1, _C1)], d1)
    pltpu.sync_copy(ones, ones_v)
    plsc.subcore_barrier()

    def chunk0(j, carry):
        pltpu.sync_copy(ones_v, cnt0.at[d0.at[j]], add=True)
        return carry

    lax.fori_loop(0, _C0H, chunk0, 0)

    def chunk1(j, carry):
        pltpu.sync_copy(ones_v, cnt1.at[d1.at[j]], add=True)
        return carry

    lax.fori_loop(0, _C1, chunk1, 0)
    plsc.subcore_barrier()

    @pl.when(c == 0)
    def _():
        pltpu.sync_copy(cnt0.at[pl.ds(s * _STR0, _STR0)],
                        cnt0_out.at[0].at[pl.ds(s * _STR0, _STR0)])
        pltpu.sync_copy(cnt1.at[pl.ds(s * _STR1, _STR1)],
                        cnt1_out.at[0].at[pl.ds(s * _STR1, _STR1)])

    @pl.when(c == 1)
    def _():
        pltpu.sync_copy(cnt0.at[pl.ds(s * _STR0, _STR0)],
                        cnt0_out.at[1].at[pl.ds(s * _STR0, _STR0)])
        pltpu.sync_copy(cnt1.at[pl.ds(s * _STR1, _STR1)],
                        cnt1_out.at[1].at[pl.ds(s * _STR1, _STR1)])


@functools.partial(
    pl.kernel,
    out_type=jax.ShapeDtypeStruct((2, _N1P, 64), jnp.float32),
    mesh=_mesh,
    compiler_params=_sc_params,
    scratch_types=[
        pltpu.VMEM((_C0, 128), jnp.int32),
        pltpu.VMEM((_C0, 128), jnp.int32),
        pltpu.VMEM((128, 64), jnp.float32),
        pltpu.VMEM_SHARED((_N1P, 64), jnp.float32),
    ],
)
def _sc_l0(xlo, xhi, srcc, dstc, zacc,
           out_sum,
           sidx, didx, rows, acc):
    c = lax.axis_index("c")
    s = lax.axis_index("s")
    stripe = s * _STR0
    pltpu.sync_copy(zacc, acc.at[pl.ds(stripe, _STR0)])
    pltpu.sync_copy(srcc.at[pl.ds(s * _C0, _C0)], sidx)
    pltpu.sync_copy(dstc.at[pl.ds(s * _C0, _C0)], didx)
    plsc.subcore_barrier()

    def chunk(j, carry):
        @pl.when(c == 0)
        def _():
            pltpu.sync_copy(xlo.at[sidx.at[j]], rows)

        @pl.when(c == 1)
        def _():
            pltpu.sync_copy(xhi.at[sidx.at[j]], rows)

        pltpu.sync_copy(rows, acc.at[didx.at[j]], add=True)
        return carry

    lax.fori_loop(0, _C0, chunk, 0)
    plsc.subcore_barrier()

    @pl.when(c == 0)
    def _():
        pltpu.sync_copy(acc.at[pl.ds(stripe, _STR0)],
                        out_sum.at[0].at[pl.ds(stripe, _STR0)])

    @pl.when(c == 1)
    def _():
        pltpu.sync_copy(acc.at[pl.ds(stripe, _STR0)],
                        out_sum.at[1].at[pl.ds(stripe, _STR0)])


@functools.partial(
    pl.kernel,
    out_type=jax.ShapeDtypeStruct((2, _N2, _HID), jnp.float32),
    mesh=_mesh,
    compiler_params=_sc_params,
    scratch_types=[
        pltpu.VMEM((_C1, 128), jnp.int32),
        pltpu.VMEM((_C1, 128), jnp.int32),
        pltpu.VMEM((128, _HID), jnp.float32),
        pltpu.VMEM_SHARED((_N2, _HID), jnp.float32),
    ],
)
def _sc_l1(h, srcc, dstc, zacc,
           out_sum,
           sidx, didx, rows, acc):
    c = lax.axis_index("c")
    s = lax.axis_index("s")
    stripe = s * _STR1
    pltpu.sync_copy(zacc, acc.at[pl.ds(stripe, _STR1)])
    base = c * (_C1 * 16) + s * _C1
    pltpu.sync_copy(srcc.at[pl.ds(base, _C1)], sidx)
    pltpu.sync_copy(dstc.at[pl.ds(base, _C1)], didx)
    plsc.subcore_barrier()

    def chunk(j, carry):
        pltpu.sync_copy(h.at[sidx.at[j]], rows)
        pltpu.sync_copy(rows, acc.at[didx.at[j]], add=True)
        return carry

    lax.fori_loop(0, _C1, chunk, 0)
    plsc.subcore_barrier()

    @pl.when(c == 0)
    def _():
        pltpu.sync_copy(acc.at[pl.ds(stripe, _STR1)],
                        out_sum.at[0].at[pl.ds(stripe, _STR1)])

    @pl.when(c == 1)
    def _():
        pltpu.sync_copy(acc.at[pl.ds(stripe, _STR1)],
                        out_sum.at[1].at[pl.ds(stripe, _STR1)])


def _tc_l0_body(sum_ref, cnt_ref, xlo_ref, xhi_ref, wl_ref, wr_ref, b_ref,
                h_ref):
    cnt = cnt_ref[0, :, 0:1] + cnt_ref[1, :, 0:1]
    inv = 1.0 / jnp.maximum(cnt, 1.0)
    slo = sum_ref[0] * inv
    shi = sum_ref[1] * inv
    wl = wl_ref[:]
    wr = wr_ref[:]
    acc = jnp.dot(slo, wl[0:64], preferred_element_type=jnp.float32)
    acc = acc + jnp.dot(shi, wl[64:128], preferred_element_type=jnp.float32)
    acc = acc + jnp.dot(xlo_ref[:], wr[0:64], preferred_element_type=jnp.float32)
    acc = acc + jnp.dot(xhi_ref[:], wr[64:128], preferred_element_type=jnp.float32)
    h_ref[:] = jnp.maximum(acc + b_ref[0:1, :], 0.0)


_TC0_ROWS = 160

_tc_l0 = pl.pallas_call(
    _tc_l0_body,
    grid=(_N1 // _TC0_ROWS,),
    in_specs=[
        pl.BlockSpec((2, _TC0_ROWS, 64), lambda i: (0, i, 0)),
        pl.BlockSpec((2, _TC0_ROWS, 16), lambda i: (0, i, 0)),
        pl.BlockSpec((_TC0_ROWS, 64), lambda i: (i, 0)),
        pl.BlockSpec((_TC0_ROWS, 64), lambda i: (i, 0)),
        pl.BlockSpec((_IN_C, _HID), lambda i: (0, 0)),
        pl.BlockSpec((_IN_C, _HID), lambda i: (0, 0)),
        pl.BlockSpec((8, _HID), lambda i: (0, 0)),
    ],
    out_specs=pl.BlockSpec((_TC0_ROWS, _HID), lambda i: (i, 0)),
    out_shape=jax.ShapeDtypeStruct((_N1, _HID), jnp.float32),
)


def _tc_l1_body(sum_ref, cnt_ref, h_ref, wl_ref, wr_ref, b_ref,
                ls_ref, z_ref):
    s2 = sum_ref[0] + sum_ref[1]
    cnt = cnt_ref[0, :, 0:1] + cnt_ref[1, :, 0:1]
    mean = s2 / jnp.maximum(cnt, 1.0)
    z = jnp.dot(mean, wl_ref[:], preferred_element_type=jnp.float32)
    z = z + jnp.dot(h_ref[:], wr_ref[:], preferred_element_type=jnp.float32)
    z = z + b_ref[0:1, :]
    m = jnp.max(z, axis=1, keepdims=True)
    lse = jnp.log(jnp.sum(jnp.exp(z - m), axis=1, keepdims=True)) + m
    z_ref[:] = z
    ls_ref[:] = z - lse


_TC1_ROWS = 256

_tc_l1 = pl.pallas_call(
    _tc_l1_body,
    grid=(_N2 // _TC1_ROWS,),
    in_specs=[
        pl.BlockSpec((2, _TC1_ROWS, _HID), lambda i: (0, i, 0)),
        pl.BlockSpec((2, _TC1_ROWS, 16), lambda i: (0, i, 0)),
        pl.BlockSpec((_TC1_ROWS, _HID), lambda i: (i, 0)),
        pl.BlockSpec((_HID, _OUT_C), lambda i: (0, 0)),
        pl.BlockSpec((_HID, _OUT_C), lambda i: (0, 0)),
        pl.BlockSpec((8, _OUT_C), lambda i: (0, 0)),
    ],
    out_specs=[
        pl.BlockSpec((_TC1_ROWS, _OUT_C), lambda i: (i, 0)),
        pl.BlockSpec((_TC1_ROWS, _OUT_C), lambda i: (i, 0)),
    ],
    out_shape=[
        jax.ShapeDtypeStruct((_N2, _OUT_C), jnp.float32),
        jax.ShapeDtypeStruct((_N2, _OUT_C), jnp.float32),
    ],
)


def kernel(x, edge_index0, edge_index1, W_l0, W_r0, b0, W_l1, W_r1, b1):
    xp = jnp.concatenate(
        [x[:_N1], jnp.zeros((_N1P - _N1, _IN_C), jnp.float32)], axis=0)
    xlo = xp[:, :64]
    xhi = xp[:, 64:]

    npad = _E0P - _E0
    padv = _N1 + (jnp.arange(npad, dtype=jnp.int32) % (_N1P - _N1))
    src0 = jnp.concatenate([edge_index0[0], padv]).reshape(_C0 * 16, 128)
    dst0 = jnp.concatenate([edge_index0[1], padv]).reshape(_C0 * 16, 128)
    src1 = edge_index1[0].reshape(_E1 // 128, 128)
    dst1 = edge_index1[1].reshape(_E1 // 128, 128)

    ones16 = jnp.ones((128, 16), jnp.float32)
    z0c = jnp.zeros((_STR0, 16), jnp.float32)
    z1c = jnp.zeros((_STR1, 16), jnp.float32)
    z0a = jnp.zeros((_STR0, 64), jnp.float32)
    z1a = jnp.zeros((_STR1, _HID), jnp.float32)

    cnt0, cnt1 = _sc_counts(dst0, dst1, z0c, z1c, ones16)
    summed0 = _sc_l0(xlo, xhi, src0, dst0, z0a)

    b0r = jnp.tile(b0[None, :], (8, 1))
    h = _tc_l0(summed0, cnt0, xlo, xhi, W_l0, W_r0, b0r)

    summed1 = _sc_l1(h, src1, dst1, z1a)

    b1r = jnp.tile(b1[None, :], (8, 1))
    ls, z = _tc_l1(summed1, cnt1, h, W_l1, W_r1, b1r)
    return (ls, z, h)
